# Initial kernel scaffold; baseline (speedup 1.0000x reference)
import functools

import jax
import jax.numpy as jnp
from jax import lax
from jax.experimental import pallas as pl
from jax.experimental.pallas import tpu as pltpu
from jax.experimental.pallas import tpu_sc as plsc

B, L = 4096, 200
E, H, C = 64, 128, 10
NC, NS, LANES = 2, 16, 16
NW = NC * NS
BPW = B // NW
NCHUNK, LC = 2, 100
ECHUNKS = E // LANES


def _pool_body(bos_h, big_h, trig_h, uni_h, bi_h, tri_h, out_h,
               idx_v, rows_v, out_v, sem):
    wid = lax.axis_index("s") * NC + lax.axis_index("c")
    base = wid * BPW

    pairs = ((bos_h, uni_h), (big_h, bi_h), (trig_h, tri_h))

    def per_b(b, carry):
        gb = base + b
        for t, (idx_h, tab_h) in enumerate(pairs):
            pltpu.sync_copy(idx_h.at[gb], idx_v)
            cps = [
                pltpu.make_async_copy(tab_h.at[idx_v.at[j]], rows_v.at[j], sem)
                for j in range(NCHUNK)
            ]
            for cp in cps:
                cp.start()
            for cp in cps:
                cp.wait()

            accs = [jnp.zeros((LANES,), jnp.float32) for _ in range(ECHUNKS)]
            for j in range(NCHUNK):
                def rbody(r, accs):
                    return tuple(
                        accs[c] + rows_v[j, r, pl.ds(LANES * c, LANES)]
                        for c in range(ECHUNKS)
                    )
                accs = lax.fori_loop(0, LC, rbody, tuple(accs))
            for c in range(ECHUNKS):
                out_v[b, pl.ds(t * E + c * LANES, LANES)] = accs[c] * (1.0 / L)
        return carry

    lax.fori_loop(0, BPW, per_b, 0)
    pltpu.sync_copy(out_v, out_h.at[pl.ds(base, BPW)])


_pool = pl.kernel(
    _pool_body,
    out_type=jax.ShapeDtypeStruct((B, 3 * E), jnp.float32),
    mesh=plsc.VectorSubcoreMesh(
        core_axis_name="c", subcore_axis_name="s",
        num_cores=NC, num_subcores=NS,
    ),
    scratch_types=[
        pltpu.VMEM((NCHUNK, LC), jnp.int32),
        pltpu.VMEM((NCHUNK, LC, E), jnp.float32),
        pltpu.VMEM((BPW, 3 * E), jnp.float32),
        pltpu.SemaphoreType.DMA,
    ],
)


def _mlp_body(x_ref, w1_ref, b1_ref, w2_ref, b2_ref, o_ref):
    h = jnp.dot(x_ref[...], w1_ref[...], preferred_element_type=jnp.float32)
    h = jnp.maximum(h + b1_ref[...], 0.0)
    o_ref[...] = jnp.dot(h, w2_ref[...], preferred_element_type=jnp.float32) + b2_ref[...]


_mlp = pl.pallas_call(
    _mlp_body,
    out_shape=jax.ShapeDtypeStruct((B, C), jnp.float32),
)


@jax.jit
def kernel(bos, bigram, trigram, uni_table, bi_table, tri_table,
           fc1_w, fc1_b, fc2_w, fc2_b):
    bos3 = bos.astype(jnp.int32).reshape(B, NCHUNK, LC)
    big3 = bigram.astype(jnp.int32).reshape(B, NCHUNK, LC)
    tri3 = trigram.astype(jnp.int32).reshape(B, NCHUNK, LC)
    pooled = _pool(bos3, big3, tri3, uni_table, bi_table, tri_table)
    return _mlp(pooled, fc1_w, fc1_b.reshape(1, H), fc2_w, fc2_b.reshape(1, C))


# speedup vs baseline: 2.3065x; 2.3065x over previous
import functools

import jax
import jax.numpy as jnp
from jax import lax
from jax.experimental import pallas as pl
from jax.experimental.pallas import tpu as pltpu
from jax.experimental.pallas import tpu_sc as plsc

B, L = 4096, 200
E, H, C = 64, 128, 10
NC, NS, LANES = 2, 16, 16
NW = NC * NS
BPW = B // NW
NCHUNK, LC = 2, 100
ECHUNKS = E // LANES


def _pool_body(bos_h, big_h, trig_h, uni_h, bi_h, tri_h, out_h,
               idx_v, rows_v, out_v, sem):
    wid = lax.axis_index("s") * NC + lax.axis_index("c")
    base = wid * BPW

    pairs = ((bos_h, uni_h), (big_h, bi_h), (trig_h, tri_h))

    def per_b(b, carry):
        gb = base + b
        for t, (idx_h, tab_h) in enumerate(pairs):
            pltpu.sync_copy(idx_h.at[gb], idx_v)
            cps = [
                pltpu.make_async_copy(tab_h.at[idx_v.at[j]], rows_v.at[j], sem)
                for j in range(NCHUNK)
            ]
            for cp in cps:
                cp.start()
            for cp in cps:
                cp.wait()

            accs = [jnp.zeros((LANES,), jnp.float32) for _ in range(ECHUNKS)]
            for j in range(NCHUNK):
                def rbody(r, accs):
                    return tuple(
                        accs[c] + rows_v[j, r, pl.ds(LANES * c, LANES)]
                        for c in range(ECHUNKS)
                    )
                accs = lax.fori_loop(0, LC, rbody, tuple(accs))
            for c in range(ECHUNKS):
                out_v[b, pl.ds(t * E + c * LANES, LANES)] = accs[c] * (1.0 / L)
        return carry

    lax.fori_loop(0, BPW, per_b, 0)
    pltpu.sync_copy(out_v, out_h.at[pl.ds(base, BPW)])


_pool = pl.kernel(
    _pool_body,
    out_type=jax.ShapeDtypeStruct((B, 3 * E), jnp.float32),
    mesh=plsc.VectorSubcoreMesh(
        core_axis_name="c", subcore_axis_name="s",
        num_cores=NC, num_subcores=NS,
    ),
    scratch_types=[
        pltpu.VMEM((NCHUNK, LC), jnp.int32),
        pltpu.VMEM((NCHUNK, LC, E), jnp.float32),
        pltpu.VMEM((BPW, 3 * E), jnp.float32),
        pltpu.SemaphoreType.DMA,
    ],
    compiler_params=pltpu.CompilerParams(use_tc_tiling_on_sc=False),
)


def _mlp_body(x_ref, w1_ref, b1_ref, w2_ref, b2_ref, o_ref):
    h = jnp.dot(x_ref[...], w1_ref[...], preferred_element_type=jnp.float32)
    h = jnp.maximum(h + b1_ref[...], 0.0)
    o_ref[...] = jnp.dot(h, w2_ref[...], preferred_element_type=jnp.float32) + b2_ref[...]


_mlp = pl.pallas_call(
    _mlp_body,
    out_shape=jax.ShapeDtypeStruct((B, C), jnp.float32),
)


@jax.jit
def kernel(bos, bigram, trigram, uni_table, bi_table, tri_table,
           fc1_w, fc1_b, fc2_w, fc2_b):
    bos3 = bos.astype(jnp.int32).reshape(B, NCHUNK, LC)
    big3 = bigram.astype(jnp.int32).reshape(B, NCHUNK, LC)
    tri3 = trigram.astype(jnp.int32).reshape(B, NCHUNK, LC)
    pooled = _pool(bos3, big3, tri3, uni_table, bi_table, tri_table)
    return _mlp(pooled, fc1_w, fc1_b.reshape(1, H), fc2_w, fc2_b.reshape(1, C))


# speedup vs baseline: 2.8510x; 1.2361x over previous
import functools

import jax
import jax.numpy as jnp
from jax import lax
from jax.experimental import pallas as pl
from jax.experimental.pallas import tpu as pltpu
from jax.experimental.pallas import tpu_sc as plsc

B, L = 4096, 200
E, H, C = 64, 128, 10
NC, NS, LANES = 2, 16, 16
NW = NC * NS
BPW = B // NW
NCHUNK, LC = 2, 100
ECHUNKS = E // LANES
HB = 64
NHALF = BPW // HB
NITEMS = 3 * NCHUNK


def _pool_body(bos_h, big_h, trig_h, uni_h, bi_h, tri_h, out_h,
               idx_v, rows_v, out_v, sem0, sem1):
    wid = lax.axis_index("s") * NC + lax.axis_index("c")
    base = wid * BPW
    tabs = (uni_h, bi_h, tri_h)
    idx_hs = (bos_h, big_h, trig_h)
    sems = (sem0, sem1)

    def copy_for(b, k):
        t, j = divmod(k, 2)
        return pltpu.make_async_copy(
            tabs[t].at[idx_v.at[t, b, j]], rows_v.at[k % 2], sems[k % 2])

    for half in range(NHALF):
        hbase = base + half * HB
        for t in range(3):
            pltpu.sync_copy(idx_hs[t].at[pl.ds(hbase, HB)], idx_v.at[t])

        copy_for(0, 0).start()

        def per_b(b, carry):
            accs = None
            for k in range(NITEMS):
                t, j = divmod(k, 2)
                if k < NITEMS - 1:
                    copy_for(b, k + 1).start()
                else:
                    @pl.when(b < HB - 1)
                    def _():
                        copy_for(b + 1, 0).start()
                copy_for(b, k).wait()

                if j == 0:
                    accs = tuple(
                        jnp.zeros((LANES,), jnp.float32) for _ in range(ECHUNKS))

                def rbody(r, accs, _slot=k % 2):
                    return tuple(
                        accs[c] + rows_v[_slot, r, pl.ds(LANES * c, LANES)]
                        for c in range(ECHUNKS)
                    )
                accs = lax.fori_loop(0, LC, rbody, accs)

                if j == 1:
                    for c in range(ECHUNKS):
                        out_v[b, pl.ds(t * E + c * LANES, LANES)] = (
                            accs[c] * (1.0 / L))
            return carry

        lax.fori_loop(0, HB, per_b, 0)
        pltpu.sync_copy(out_v, out_h.at[pl.ds(hbase, HB)])


_pool = pl.kernel(
    _pool_body,
    out_type=jax.ShapeDtypeStruct((B, 3 * E), jnp.float32),
    mesh=plsc.VectorSubcoreMesh(
        core_axis_name="c", subcore_axis_name="s",
        num_cores=NC, num_subcores=NS,
    ),
    scratch_types=[
        pltpu.VMEM((3, HB, NCHUNK, LC), jnp.int32),
        pltpu.VMEM((2, LC, E), jnp.float32),
        pltpu.VMEM((HB, 3 * E), jnp.float32),
        pltpu.SemaphoreType.DMA,
        pltpu.SemaphoreType.DMA,
    ],
    compiler_params=pltpu.CompilerParams(use_tc_tiling_on_sc=False),
)


def _mlp_body(x_ref, w1_ref, b1_ref, w2_ref, b2_ref, o_ref):
    h = jnp.dot(x_ref[...], w1_ref[...], preferred_element_type=jnp.float32)
    h = jnp.maximum(h + b1_ref[...], 0.0)
    o_ref[...] = jnp.dot(h, w2_ref[...], preferred_element_type=jnp.float32) + b2_ref[...]


_mlp = pl.pallas_call(
    _mlp_body,
    out_shape=jax.ShapeDtypeStruct((B, C), jnp.float32),
)


@jax.jit
def kernel(bos, bigram, trigram, uni_table, bi_table, tri_table,
           fc1_w, fc1_b, fc2_w, fc2_b):
    bos3 = bos.astype(jnp.int32).reshape(B, NCHUNK, LC)
    big3 = bigram.astype(jnp.int32).reshape(B, NCHUNK, LC)
    tri3 = trigram.astype(jnp.int32).reshape(B, NCHUNK, LC)
    pooled = _pool(bos3, big3, tri3, uni_table, bi_table, tri_table)
    return _mlp(pooled, fc1_w, fc1_b.reshape(1, H), fc2_w, fc2_b.reshape(1, C))


# speedup vs baseline: 3.1236x; 1.0956x over previous
import functools

import jax
import jax.numpy as jnp
from jax import lax
from jax.experimental import pallas as pl
from jax.experimental.pallas import tpu as pltpu
from jax.experimental.pallas import tpu_sc as plsc

B, L = 4096, 200
E, H, C = 64, 128, 10
NC, NS, LANES = 2, 16, 16
NW = NC * NS
BPW = B // NW
NCHUNK, LC = 2, 100
ECHUNKS = E // LANES
HB = 64
NHALF = BPW // HB
NITEMS = 3 * NCHUNK


NSLOT = 3
LOOKAHEAD = 2


def _pool_body(bos_h, big_h, trig_h, uni_h, bi_h, tri_h, out_h,
               idx_v, rows_v, out_v, sem0, sem1, sem2):
    wid = lax.axis_index("s") * NC + lax.axis_index("c")
    base = wid * BPW
    tabs = (uni_h, bi_h, tri_h)
    idx_hs = (bos_h, big_h, trig_h)
    sems = (sem0, sem1, sem2)

    def copy_for(b, k):
        t, j = divmod(k, 2)
        return pltpu.make_async_copy(
            tabs[t].at[idx_v.at[t, b, j]], rows_v.at[k % NSLOT], sems[k % NSLOT])

    for half in range(NHALF):
        hbase = base + half * HB
        for t in range(3):
            pltpu.sync_copy(idx_hs[t].at[pl.ds(hbase, HB)], idx_v.at[t])

        for k in range(LOOKAHEAD):
            copy_for(0, k).start()

        def per_b(b, carry):
            accs = None
            for k in range(NITEMS):
                t, j = divmod(k, 2)
                ka = k + LOOKAHEAD
                if ka < NITEMS:
                    copy_for(b, ka).start()
                else:
                    @pl.when(b < HB - 1)
                    def _():
                        copy_for(b + 1, ka - NITEMS).start()
                copy_for(b, k).wait()

                if j == 0:
                    accs = tuple(
                        jnp.zeros((LANES,), jnp.float32) for _ in range(ECHUNKS))

                @plsc.parallel_loop(0, LC, unroll=4, carry=accs)
                def accs(r, accs, _slot=k % NSLOT):
                    return tuple(
                        accs[c] + rows_v[_slot, r, pl.ds(LANES * c, LANES)]
                        for c in range(ECHUNKS)
                    )

                if j == 1:
                    for c in range(ECHUNKS):
                        out_v[b, pl.ds(t * E + c * LANES, LANES)] = (
                            accs[c] * (1.0 / L))
            return carry

        lax.fori_loop(0, HB, per_b, 0)
        pltpu.sync_copy(out_v, out_h.at[pl.ds(hbase, HB)])


_pool = pl.kernel(
    _pool_body,
    out_type=jax.ShapeDtypeStruct((B, 3 * E), jnp.float32),
    mesh=plsc.VectorSubcoreMesh(
        core_axis_name="c", subcore_axis_name="s",
        num_cores=NC, num_subcores=NS,
    ),
    scratch_types=[
        pltpu.VMEM((3, HB, NCHUNK, LC), jnp.int32),
        pltpu.VMEM((NSLOT, LC, E), jnp.float32),
        pltpu.VMEM((HB, 3 * E), jnp.float32),
        pltpu.SemaphoreType.DMA,
        pltpu.SemaphoreType.DMA,
        pltpu.SemaphoreType.DMA,
    ],
    compiler_params=pltpu.CompilerParams(use_tc_tiling_on_sc=False),
)


def _mlp_body(x_ref, w1_ref, b1_ref, w2_ref, b2_ref, o_ref):
    h = jnp.dot(x_ref[...], w1_ref[...], preferred_element_type=jnp.float32)
    h = jnp.maximum(h + b1_ref[...], 0.0)
    o_ref[...] = jnp.dot(h, w2_ref[...], preferred_element_type=jnp.float32) + b2_ref[...]


_mlp = pl.pallas_call(
    _mlp_body,
    out_shape=jax.ShapeDtypeStruct((B, C), jnp.float32),
)


@jax.jit
def kernel(bos, bigram, trigram, uni_table, bi_table, tri_table,
           fc1_w, fc1_b, fc2_w, fc2_b):
    bos3 = bos.astype(jnp.int32).reshape(B, NCHUNK, LC)
    big3 = bigram.astype(jnp.int32).reshape(B, NCHUNK, LC)
    tri3 = trigram.astype(jnp.int32).reshape(B, NCHUNK, LC)
    pooled = _pool(bos3, big3, tri3, uni_table, bi_table, tri_table)
    return _mlp(pooled, fc1_w, fc1_b.reshape(1, H), fc2_w, fc2_b.reshape(1, C))


# speedup vs baseline: 3.1482x; 1.0079x over previous
import functools

import jax
import jax.numpy as jnp
from jax import lax
from jax.experimental import pallas as pl
from jax.experimental.pallas import tpu as pltpu
from jax.experimental.pallas import tpu_sc as plsc

B, L = 4096, 200
E, H, C = 64, 128, 10
NC, NS, LANES = 2, 16, 16
NW = NC * NS
BPW = B // NW
NCHUNK = 2
LCS = (104, 96)
LOFF = (0, 104)
LCMAX = 104
ECHUNKS = E // LANES
HB = 64
NHALF = BPW // HB
NITEMS = 3 * NCHUNK


NSLOT = 3
LOOKAHEAD = 2


def _pool_body(bos_h, big_h, trig_h, uni_h, bi_h, tri_h, out_h,
               idx_v, rows_v, out_v, sem0, sem1, sem2):
    wid = lax.axis_index("s") * NC + lax.axis_index("c")
    base = wid * BPW
    tabs = (uni_h, bi_h, tri_h)
    idx_hs = (bos_h, big_h, trig_h)
    sems = (sem0, sem1, sem2)

    def copy_for(b, k):
        t, j = divmod(k, 2)
        return pltpu.make_async_copy(
            tabs[t].at[idx_v.at[t, b, pl.ds(LOFF[j], LCS[j])]],
            rows_v.at[k % NSLOT, pl.ds(0, LCS[j])], sems[k % NSLOT])

    for half in range(NHALF):
        hbase = base + half * HB
        for t in range(3):
            pltpu.sync_copy(idx_hs[t].at[pl.ds(hbase, HB)], idx_v.at[t])

        for k in range(LOOKAHEAD):
            copy_for(0, k).start()

        def per_b(b, carry):
            accs = None
            for k in range(NITEMS):
                t, j = divmod(k, 2)
                ka = k + LOOKAHEAD
                if ka < NITEMS:
                    copy_for(b, ka).start()
                else:
                    @pl.when(b < HB - 1)
                    def _():
                        copy_for(b + 1, ka - NITEMS).start()
                copy_for(b, k).wait()

                if j == 0:
                    accs = tuple(
                        jnp.zeros((LANES,), jnp.float32) for _ in range(ECHUNKS))

                @plsc.parallel_loop(0, LCS[j], unroll=4, carry=accs)
                def accs(r, accs, _slot=k % NSLOT):
                    return tuple(
                        accs[c] + rows_v[_slot, r, pl.ds(LANES * c, LANES)]
                        for c in range(ECHUNKS)
                    )

                if j == 1:
                    for c in range(ECHUNKS):
                        out_v[b, pl.ds(t * E + c * LANES, LANES)] = (
                            accs[c] * (1.0 / L))
            return carry

        lax.fori_loop(0, HB, per_b, 0)
        pltpu.sync_copy(out_v, out_h.at[pl.ds(hbase, HB)])


_pool = pl.kernel(
    _pool_body,
    out_type=jax.ShapeDtypeStruct((B, 3 * E), jnp.float32),
    mesh=plsc.VectorSubcoreMesh(
        core_axis_name="c", subcore_axis_name="s",
        num_cores=NC, num_subcores=NS,
    ),
    scratch_types=[
        pltpu.VMEM((3, HB, L), jnp.int32),
        pltpu.VMEM((NSLOT, LCMAX, E), jnp.float32),
        pltpu.VMEM((HB, 3 * E), jnp.float32),
        pltpu.SemaphoreType.DMA,
        pltpu.SemaphoreType.DMA,
        pltpu.SemaphoreType.DMA,
    ],
    compiler_params=pltpu.CompilerParams(use_tc_tiling_on_sc=False),
)


def _mlp_body(x_ref, w1_ref, b1_ref, w2_ref, b2_ref, o_ref):
    h = jnp.dot(x_ref[...], w1_ref[...], preferred_element_type=jnp.float32)
    h = jnp.maximum(h + b1_ref[...], 0.0)
    o_ref[...] = jnp.dot(h, w2_ref[...], preferred_element_type=jnp.float32) + b2_ref[...]


_mlp = pl.pallas_call(
    _mlp_body,
    out_shape=jax.ShapeDtypeStruct((B, C), jnp.float32),
)


@jax.jit
def kernel(bos, bigram, trigram, uni_table, bi_table, tri_table,
           fc1_w, fc1_b, fc2_w, fc2_b):
    pooled = _pool(bos, bigram, trigram, uni_table, bi_table, tri_table)
    return _mlp(pooled, fc1_w, fc1_b.reshape(1, H), fc2_w, fc2_b.reshape(1, C))
